# baseline (speedup 1.0000x reference)
import jax
import jax.numpy as jnp
from jax.experimental import pallas as pl
from jax.experimental.pallas import tpu as pltpu

_NUM_CHUNKS = 32


def kernel(input_ids, positional_encoding_table):
    seq_len = input_ids.shape[1]
    model_dim = positional_encoding_table.shape[1]
    chunk = max(1, seq_len // _NUM_CHUNKS)
    nchunks = (seq_len + chunk - 1) // chunk

    def body(table_ref, out_ref, sems):
        for i in range(nchunks):
            lo = i * chunk
            sz = min(chunk, seq_len - lo)
            pltpu.make_async_copy(
                table_ref.at[pl.ds(lo, sz), :],
                out_ref.at[pl.ds(lo, sz), :],
                sems.at[i],
            ).start()
        for i in range(nchunks):
            lo = i * chunk
            sz = min(chunk, seq_len - lo)
            pltpu.make_async_copy(
                table_ref.at[pl.ds(lo, sz), :],
                out_ref.at[pl.ds(lo, sz), :],
                sems.at[i],
            ).wait()

    return pl.pallas_call(
        body,
        out_shape=jax.ShapeDtypeStruct((seq_len, model_dim),
                                       positional_encoding_table.dtype),
        in_specs=[pl.BlockSpec(memory_space=pl.ANY)],
        out_specs=pl.BlockSpec(memory_space=pl.ANY),
        scratch_shapes=[pltpu.SemaphoreType.DMA((nchunks,))],
    )(positional_encoding_table)


# speedup vs baseline: 46.6997x; 46.6997x over previous
import jax
import jax.numpy as jnp
from jax.experimental import pallas as pl
from jax.experimental.pallas import tpu as pltpu

_BLOCK_ROWS = 512


def kernel(input_ids, positional_encoding_table):
    seq_len = input_ids.shape[1]
    model_dim = positional_encoding_table.shape[1]
    grid = (seq_len // _BLOCK_ROWS,)

    def body(table_ref, out_ref):
        out_ref[...] = table_ref[...]

    return pl.pallas_call(
        body,
        out_shape=jax.ShapeDtypeStruct((seq_len, model_dim),
                                       positional_encoding_table.dtype),
        grid=grid,
        in_specs=[pl.BlockSpec((_BLOCK_ROWS, model_dim), lambda i: (i, 0))],
        out_specs=pl.BlockSpec((_BLOCK_ROWS, model_dim), lambda i: (i, 0)),
    )(positional_encoding_table)


# speedup vs baseline: 48.7474x; 1.0438x over previous
import jax
import jax.numpy as jnp
from jax.experimental import pallas as pl
from jax.experimental.pallas import tpu as pltpu

_BLOCK_ROWS = 1024


def kernel(input_ids, positional_encoding_table):
    seq_len = input_ids.shape[1]
    model_dim = positional_encoding_table.shape[1]
    grid = (seq_len // _BLOCK_ROWS,)

    def body(table_ref, out_ref):
        out_ref[...] = table_ref[...]

    return pl.pallas_call(
        body,
        out_shape=jax.ShapeDtypeStruct((seq_len, model_dim),
                                       positional_encoding_table.dtype),
        grid=grid,
        in_specs=[pl.BlockSpec((_BLOCK_ROWS, model_dim), lambda i: (i, 0))],
        out_specs=pl.BlockSpec((_BLOCK_ROWS, model_dim), lambda i: (i, 0)),
    )(positional_encoding_table)
